# baseline (speedup 1.0000x reference)
import functools

import jax
import jax.numpy as jnp
from jax import lax
from jax.experimental import pallas as pl
from jax.experimental.pallas import tpu as pltpu
from jax.experimental.pallas import tpu_sc as plsc

NUM_CORES = 2
NUM_SUBCORES = 16
NUM_WORKERS = NUM_CORES * NUM_SUBCORES
LANES = 16
BATCH = 16384
DIM = 32
BPW = BATCH // NUM_WORKERS
GROUPS = BPW // LANES


def _sc_body(uid_hbm, mid_hbm, uemb_hbm, memb_hbm, ubias_hbm, mbias_hbm,
             w_hbm, b_hbm, out_hbm,
             uid_v, mid_v, urows_v, mrows_v, ub_v, mb_v, w_v, b_v, out_v,
             sem_u, sem_m, sem_ub, sem_mb):
    wid = lax.axis_index("s") * NUM_CORES + lax.axis_index("c")
    base = wid * BPW

    pltpu.sync_copy(uid_hbm.at[pl.ds(base, BPW)], uid_v)
    pltpu.sync_copy(mid_hbm.at[pl.ds(base, BPW)], mid_v)

    cu = pltpu.async_copy(uemb_hbm.at[uid_v], urows_v, sem_u)
    cm = pltpu.async_copy(memb_hbm.at[mid_v], mrows_v, sem_m)
    cub = pltpu.async_copy(ubias_hbm.at[uid_v], ub_v, sem_ub)
    cmb = pltpu.async_copy(mbias_hbm.at[mid_v], mb_v, sem_mb)

    pltpu.sync_copy(w_hbm, w_v)
    pltpu.sync_copy(b_hbm, b_v)

    w0 = w_v[pl.ds(0, LANES)]
    w1 = w_v[pl.ds(LANES, LANES)]
    out_bias = b_v[pl.ds(0, LANES)][0]

    cu.wait()
    cm.wait()
    cub.wait()
    cmb.wait()

    def group(g, carry):
        gbase = g * LANES
        row_idx = gbase + lax.iota(jnp.int32, LANES)
        ubmb = ub_v[pl.ds(gbase, LANES)] + mb_v[pl.ds(gbase, LANES)]
        acc = jnp.zeros((LANES,), jnp.float32) + out_bias
        for d in range(DIM):
            col = jnp.full((LANES,), d, jnp.int32)
            u_c = plsc.load_gather(urows_v, [row_idx, col])
            m_c = plsc.load_gather(mrows_v, [row_idx, col])
            w_d = (w0 if d < LANES else w1)[d % LANES]
            acc = acc + (u_c * m_c + ubmb) * w_d
        out_v[pl.ds(gbase, LANES)] = acc
        return carry

    lax.fori_loop(0, GROUPS, group, 0)
    pltpu.sync_copy(out_v, out_hbm.at[pl.ds(base, BPW)])


@jax.jit
def _run(user_ids, movie_tags, user_emb, movie_emb, user_bias_flat,
         movie_bias_flat, out_w_flat, out_b_pad):
    mesh = plsc.VectorSubcoreMesh(core_axis_name="c", subcore_axis_name="s",
                                  num_cores=NUM_CORES, num_subcores=NUM_SUBCORES)
    f = pl.kernel(
        _sc_body,
        out_type=jax.ShapeDtypeStruct((BATCH,), jnp.float32),
        mesh=mesh,
        scratch_types=[
            pltpu.VMEM((BPW,), jnp.int32),
            pltpu.VMEM((BPW,), jnp.int32),
            pltpu.VMEM((BPW, DIM), jnp.float32),
            pltpu.VMEM((BPW, DIM), jnp.float32),
            pltpu.VMEM((BPW,), jnp.float32),
            pltpu.VMEM((BPW,), jnp.float32),
            pltpu.VMEM((DIM,), jnp.float32),
            pltpu.VMEM((LANES,), jnp.float32),
            pltpu.VMEM((BPW,), jnp.float32),
            pltpu.SemaphoreType.DMA,
            pltpu.SemaphoreType.DMA,
            pltpu.SemaphoreType.DMA,
            pltpu.SemaphoreType.DMA,
        ],
        compiler_params=pltpu.CompilerParams(needs_layout_passes=False,
                                             use_tc_tiling_on_sc=False),
    )
    return f(user_ids, movie_tags, user_emb, movie_emb, user_bias_flat,
             movie_bias_flat, out_w_flat, out_b_pad)


def kernel(user_ids, movie_tags, user_emb, movie_emb, user_bias, movie_bias,
           out_w, out_b):
    out = _run(
        user_ids.astype(jnp.int32),
        movie_tags.astype(jnp.int32),
        user_emb,
        movie_emb,
        jnp.reshape(user_bias, (-1,)),
        jnp.reshape(movie_bias, (-1,)),
        jnp.reshape(out_w, (-1,)),
        jnp.pad(jnp.reshape(out_b, (-1,)), (0, LANES - 1)),
    )
    return jnp.reshape(out, (BATCH, 1))
